# baseline (speedup 1.0000x reference)
import jax
import jax.numpy as jnp
from jax import lax
from jax.experimental import pallas as pl
from jax.experimental.pallas import tpu as pltpu
from jax.experimental.pallas import tpu_sc as plsc

NUM_CORES = 2
NUM_SUBCORES = 16
LANES = 16
NUM_WORKERS = NUM_CORES * NUM_SUBCORES

BATCH = 16384
EMBED_DIM = 64
ROWS_PER_WORKER = BATCH // NUM_WORKERS
CHUNK = 128
NUM_CHUNKS = ROWS_PER_WORKER // CHUNK
NUM_GROUPS = ROWS_PER_WORKER // LANES

_EXPONENT = -0.49
_LN2 = 0.6931471805599453
_GRAVITATION = 1e-07


def _sc_body(qtab, ctab, qids, cids, out, qidx_v, cidx_v, qrows, crows,
             outbuf, sem):
    wid = lax.axis_index("s") * NUM_CORES + lax.axis_index("c")

    pltpu.sync_copy(qids.at[wid], qidx_v)
    pltpu.sync_copy(cids.at[wid], cidx_v)

    handles = []
    for j in range(NUM_CHUNKS):
        dst_q = qrows.at[pl.ds(j * CHUNK, CHUNK)]
        dst_c = crows.at[pl.ds(j * CHUNK, CHUNK)]
        handles.append(pltpu.async_copy(qtab.at[qidx_v.at[j]], dst_q, sem))
        handles.append(pltpu.async_copy(ctab.at[cidx_v.at[j]], dst_c, sem))
    for h in handles:
        h.wait()

    lane = lax.iota(jnp.int32, LANES)

    def group_body(g, carry):
        cacc, gacc = carry
        rowv = g * LANES + lane

        def dim_body(d, c3):
            dot, qn, cn = c3
            colv = jnp.full((LANES,), d, dtype=jnp.int32)
            qv = plsc.load_gather(qrows, [rowv, colv])
            cv = plsc.load_gather(crows, [rowv, colv])
            return dot + qv * cv, qn + qv * qv, cn + cv * cv

        zeros = jnp.zeros((LANES,), jnp.float32)
        dot, qn, cn = lax.fori_loop(
            0, EMBED_DIM, dim_body, (zeros, zeros, zeros), unroll=8)

        prod = qn * cn
        bits = plsc.bitcast(prod, jnp.int32)
        e = (bits >> 23) - 127
        mbits = (bits & 0x007FFFFF) | 0x3F800000
        m = plsc.bitcast(mbits, jnp.float32)
        t = (m - 1.0) / (m + 1.0)
        t2 = t * t
        poly = ((((t2 / 9.0 + 1.0 / 7.0) * t2 + 0.2) * t2 + 1.0 / 3.0)
                * t2 + 1.0)
        ln_prod = e.astype(jnp.float32) * _LN2 + 2.0 * t * poly
        pw = jnp.exp(_EXPONENT * ln_prod)

        return cacc + dot * pw, gacc + (qn + cn)

    zeros = jnp.zeros((LANES,), jnp.float32)
    cacc, gacc = lax.fori_loop(0, NUM_GROUPS, group_body, (zeros, zeros))

    outbuf[0, :] = cacc
    outbuf[1, :] = gacc
    pltpu.sync_copy(outbuf, out.at[wid])


@jax.jit
def _run(query_table, candidate_table, qids_r, cids_r):
    mesh = plsc.VectorSubcoreMesh(
        core_axis_name="c", subcore_axis_name="s",
        num_cores=NUM_CORES, num_subcores=NUM_SUBCORES)
    parts = pl.kernel(
        _sc_body,
        out_type=jax.ShapeDtypeStruct((NUM_WORKERS, 2, LANES), jnp.float32),
        mesh=mesh,
        scratch_types=[
            pltpu.MemorySpace.VMEM((NUM_CHUNKS, CHUNK), jnp.int32),
            pltpu.MemorySpace.VMEM((NUM_CHUNKS, CHUNK), jnp.int32),
            pltpu.MemorySpace.VMEM((ROWS_PER_WORKER, EMBED_DIM), jnp.float32),
            pltpu.MemorySpace.VMEM((ROWS_PER_WORKER, EMBED_DIM), jnp.float32),
            pltpu.MemorySpace.VMEM((2, LANES), jnp.float32),
            pltpu.SemaphoreType.DMA,
        ],
        compiler_params=pltpu.CompilerParams(
            needs_layout_passes=False, use_tc_tiling_on_sc=False),
    )(query_table, candidate_table, qids_r, cids_r)
    cos_loss = -jnp.sum(parts[:, 0, :])
    grav_loss = jnp.sum(parts[:, 1, :])
    return cos_loss + _GRAVITATION * grav_loss


def kernel(query_table, candidate_table, query_ids, candidate_ids):
    qids_r = query_ids.astype(jnp.int32).reshape(NUM_WORKERS, NUM_CHUNKS, CHUNK)
    cids_r = candidate_ids.astype(jnp.int32).reshape(
        NUM_WORKERS, NUM_CHUNKS, CHUNK)
    return _run(query_table, candidate_table, qids_r, cids_r)
